# Initial kernel scaffold; baseline (speedup 1.0000x reference)
import jax
import jax.numpy as jnp
from jax.experimental import pallas as pl


def kernel(x, edge_index, pred_edge_index, conv1_W, conv1_b, conv2_W, conv2_b, lp_W1, lp_b1, lp_W2, lp_b2):
    raise NotImplementedError("write your pallas kernel here")


# speedup vs baseline: 1.2662x; 1.2662x over previous
import functools

import jax
import jax.numpy as jnp
from jax.experimental import pallas as pl
from jax.experimental.pallas import tpu as pltpu

N_NODES = 10000
IN_CH = 256
HID_CH = 512
OUT_CH = 256


def _decode_body(u_ref, v_ref, w2_ref, b2_ref, o_ref):
    h = jnp.maximum(u_ref[...] + v_ref[...], 0.0)
    s = jnp.sum(h * w2_ref[...], axis=1, keepdims=True) + b2_ref[0, 0]
    o_ref[...] = jax.nn.sigmoid(s)


def _decode(ug, vg, w2, b2):
    P = ug.shape[0]
    BLK = 2000
    grid = (P // BLK,)
    return pl.pallas_call(
        _decode_body,
        grid=grid,
        in_specs=[
            pl.BlockSpec((BLK, HID_CH), lambda i: (i, 0)),
            pl.BlockSpec((BLK, HID_CH), lambda i: (i, 0)),
            pl.BlockSpec((1, HID_CH), lambda i: (0, 0)),
            pl.BlockSpec((1, 1), lambda i: (0, 0), memory_space=pltpu.SMEM),
        ],
        out_specs=pl.BlockSpec((BLK, 1), lambda i: (i, 0)),
        out_shape=jax.ShapeDtypeStruct((P, 1), jnp.float32),
    )(ug, vg, w2, b2)


def kernel(x, edge_index, pred_edge_index, conv1_W, conv1_b, conv2_W, conv2_b,
           lp_W1, lp_b1, lp_W2, lp_b2):
    row = edge_index[0].astype(jnp.int32)
    col = edge_index[1].astype(jnp.int32)
    loop = jnp.arange(N_NODES, dtype=jnp.int32)
    row = jnp.concatenate([row, loop])
    col = jnp.concatenate([col, loop])

    deg = jnp.zeros((N_NODES,), jnp.float32).at[col].add(1.0)
    dis = jax.lax.rsqrt(deg)

    xs1 = dis[:, None] * (x @ conv1_W)
    acc1 = jnp.zeros((N_NODES, HID_CH), jnp.float32).at[col].add(
        jnp.take(xs1, row, axis=0))
    z = jax.nn.relu(dis[:, None] * acc1 + conv1_b)

    xs2 = dis[:, None] * (z @ conv2_W)
    acc2 = jnp.zeros((N_NODES, OUT_CH), jnp.float32).at[col].add(
        jnp.take(xs2, row, axis=0))
    z2 = dis[:, None] * acc2 + conv2_b

    u = z2 @ lp_W1[:OUT_CH] + lp_b1
    v = z2 @ lp_W1[OUT_CH:]
    rp = pred_edge_index[0].astype(jnp.int32)
    cp = pred_edge_index[1].astype(jnp.int32)
    ug = jnp.take(u, rp, axis=0)
    vg = jnp.take(v, cp, axis=0)
    return _decode(ug, vg, lp_W2.reshape(1, HID_CH), lp_b2.reshape(1, 1))


# speedup vs baseline: 1.2663x; 1.0001x over previous
import functools

import jax
import jax.numpy as jnp
from jax import lax
from jax.experimental import pallas as pl
from jax.experimental.pallas import tpu as pltpu
from jax.experimental.pallas import tpu_sc as plsc

N_NODES = 10000
IN_CH = 256
HID_CH = 512
OUT_CH = 256

E_RAW = 160000
E_TOT = 172032
NB_E = 84
NPAD = 10240
TRASH = 10000

_SC_MESH = plsc.VectorSubcoreMesh(core_axis_name="c", subcore_axis_name="s")


CHW = 128


def _spmm_body(C, xs_hbm, rowadj_hbm, col_hbm, zeros_hbm, out_hbm,
               rowadj_v, col_v, rows0, rows1, sem0, sem1, acc):
    half = C // 2
    cid = lax.axis_index("c")
    sid = lax.axis_index("s")
    pltpu.sync_copy(col_hbm.at[sid], col_v)
    bufs = (rows0, rows1)
    sems = (sem0, sem1)
    for t in range(half):
        chunk = cid * half + t
        pltpu.sync_copy(zeros_hbm.at[pl.ds(sid * 640, 640)],
                        acc.at[pl.ds(sid * 640, 640)])
        pltpu.sync_copy(rowadj_hbm.at[chunk].at[sid], rowadj_v)
        plsc.subcore_barrier()
        cps = [None, None]
        cps[0] = pltpu.async_copy(xs_hbm.at[rowadj_v.at[0]], bufs[0], sems[0])
        for j in range(NB_E):
            cur = j % 2
            nxt = 1 - cur
            if j + 1 < NB_E:
                cps[nxt] = pltpu.async_copy(xs_hbm.at[rowadj_v.at[j + 1]],
                                            bufs[nxt], sems[nxt])
            cps[cur].wait()
            pltpu.sync_copy(bufs[cur], acc.at[col_v.at[j]], add=True)
        plsc.subcore_barrier()
        pltpu.sync_copy(acc.at[pl.ds(sid * 640, 640)],
                        out_hbm.at[chunk].at[pl.ds(sid * 640, 640)])
        plsc.subcore_barrier()


def _spmm(xs, rowadj, col_sc, zeros, C):
    f = pl.kernel(
        functools.partial(_spmm_body, C),
        out_type=jax.ShapeDtypeStruct((C, NPAD, CHW), jnp.float32),
        mesh=_SC_MESH,
        scratch_types=[
            pltpu.VMEM((NB_E, 128), jnp.int32),
            pltpu.VMEM((NB_E, 128), jnp.int32),
            pltpu.VMEM((128, CHW), jnp.float32),
            pltpu.VMEM((128, CHW), jnp.float32),
            pltpu.SemaphoreType.DMA,
            pltpu.SemaphoreType.DMA,
            pltpu.VMEM_SHARED((NPAD, CHW), jnp.float32),
        ],
    )
    return f(xs, rowadj, col_sc, zeros)


def _chunked(a, C):
    return a.reshape(N_NODES, C, CHW).transpose(1, 0, 2).reshape(C * N_NODES, CHW)


def _unchunked(a, C):
    return a[:, :N_NODES].transpose(1, 0, 2).reshape(N_NODES, C * CHW)


def _decode_body(u_ref, v_ref, w2_ref, b2_ref, o_ref):
    h = jnp.maximum(u_ref[...] + v_ref[...], 0.0)
    s = jnp.sum(h * w2_ref[...], axis=1, keepdims=True) + b2_ref[0, 0]
    o_ref[...] = jax.nn.sigmoid(s)


def _decode(ug, vg, w2, b2):
    P = ug.shape[0]
    BLK = 2000
    return pl.pallas_call(
        _decode_body,
        grid=(P // BLK,),
        in_specs=[
            pl.BlockSpec((BLK, HID_CH), lambda i: (i, 0)),
            pl.BlockSpec((BLK, HID_CH), lambda i: (i, 0)),
            pl.BlockSpec((1, HID_CH), lambda i: (0, 0)),
            pl.BlockSpec((1, 1), lambda i: (0, 0), memory_space=pltpu.SMEM),
        ],
        out_specs=pl.BlockSpec((BLK, 1), lambda i: (i, 0)),
        out_shape=jax.ShapeDtypeStruct((P, 1), jnp.float32),
    )(ug, vg, w2, b2)


def kernel(x, edge_index, pred_edge_index, conv1_W, conv1_b, conv2_W, conv2_b,
           lp_W1, lp_b1, lp_W2, lp_b2):
    loop = jnp.arange(N_NODES, dtype=jnp.int32)
    row = jnp.concatenate([edge_index[0].astype(jnp.int32), loop,
                           jnp.zeros((E_TOT - E_RAW - N_NODES,), jnp.int32)])
    col = jnp.concatenate([edge_index[1].astype(jnp.int32), loop,
                           jnp.full((E_TOT - E_RAW - N_NODES,), TRASH, jnp.int32)])
    col_sc = col.reshape(16, NB_E, 128)
    nc1 = HID_CH // CHW
    rowadj1 = (row.reshape(1, 16, NB_E, 128)
               + (jnp.arange(nc1, dtype=jnp.int32) * N_NODES)[:, None, None, None])
    zeros = jnp.zeros((NPAD, CHW), jnp.float32)

    deg = jnp.zeros((N_NODES,), jnp.float32).at[col[:E_RAW + N_NODES]].add(1.0)
    dis = jax.lax.rsqrt(deg)

    rc = row[:E_RAW + N_NODES]
    cc = col[:E_RAW + N_NODES]
    xs1 = dis[:, None] * (x @ conv1_W)
    acc1 = jnp.zeros((N_NODES, HID_CH), jnp.float32).at[cc].add(jnp.take(xs1, rc, axis=0))
    z = jax.nn.relu(dis[:, None] * acc1 + conv1_b)

    xs2 = dis[:, None] * (z @ conv2_W)
    acc2 = jnp.zeros((N_NODES, OUT_CH), jnp.float32).at[cc].add(jnp.take(xs2, rc, axis=0))
    z2 = dis[:, None] * acc2 + conv2_b

    u = z2 @ lp_W1[:OUT_CH] + lp_b1
    v = z2 @ lp_W1[OUT_CH:]
    rp = pred_edge_index[0].astype(jnp.int32)
    cp = pred_edge_index[1].astype(jnp.int32)
    ug = jnp.take(u, rp, axis=0)
    vg = jnp.take(v, cp, axis=0)
    return _decode(ug, vg, lp_W2.reshape(1, HID_CH), lp_b2.reshape(1, 1))


# speedup vs baseline: 3.6552x; 2.8865x over previous
import functools

import jax
import jax.numpy as jnp
from jax import lax
from jax.experimental import pallas as pl
from jax.experimental.pallas import tpu as pltpu
from jax.experimental.pallas import tpu_sc as plsc

N_NODES = 10000
IN_CH = 256
HID_CH = 512
OUT_CH = 256

E_RAW = 160000
E_REAL = E_RAW + N_NODES
E_TOT = 172032
NBA = 84
HALF_N = 5120
ACC_ROWS = 5248
TRASH_L = 5120
PAD_COL = 10240
HIST_ROWS = 10368

P_RAW = 100000
P_TOT = 100352
PPW = 3136
PB = 32
NPB = PPW // PB

_SC_MESH = plsc.VectorSubcoreMesh(core_axis_name="c", subcore_axis_name="s")


def _iota16():
    return lax.broadcasted_iota(jnp.int32, (16,), 0)


def _lane_shuffle(x, idx):
    dn = lax.GatherDimensionNumbers(offset_dims=(), collapsed_slice_dims=(0,),
                                    start_index_map=(0,))
    return lax.gather(x, idx[:, None], dn, slice_sizes=(1,),
                      mode=lax.GatherScatterMode.PROMISE_IN_BOUNDS)


def _deg_body(col_hbm, ones_hbm, zeros_hbm, hist_hbm, col_v, ones_v, hist_s):
    cid = lax.axis_index("c")
    sid = lax.axis_index("s")
    wid = cid * 16 + sid
    pltpu.sync_copy(col_hbm.at[wid], col_v)
    pltpu.sync_copy(ones_hbm, ones_v)
    pltpu.sync_copy(zeros_hbm.at[pl.ds(sid * 648, 648)],
                    hist_s.at[pl.ds(sid * 648, 648)])
    plsc.subcore_barrier()
    for j in range(E_TOT // 32 // 128):
        pltpu.sync_copy(ones_v, hist_s.at[col_v.at[j]], add=True)
    plsc.subcore_barrier()
    pltpu.sync_copy(hist_s.at[pl.ds(sid * 648, 648)],
                    hist_hbm.at[cid].at[pl.ds(sid * 648, 648)])


def _deg(col32, ones16, zeros16):
    f = pl.kernel(
        _deg_body,
        out_type=jax.ShapeDtypeStruct((2, HIST_ROWS, 16), jnp.float32),
        mesh=_SC_MESH,
        scratch_types=[
            pltpu.VMEM((E_TOT // 32 // 128, 128), jnp.int32),
            pltpu.VMEM((128, 16), jnp.float32),
            pltpu.VMEM_SHARED((HIST_ROWS, 16), jnp.float32),
        ],
    )
    return f(col32, ones16, zeros16)


def _spmm_body(C, xs_hbm, rowadj_hbm, colloc_hbm, zacc_hbm, out_hbm,
               radj_v, col_v, rows0, rows1, sem0, sem1, acc):
    cid = lax.axis_index("c")
    sid = lax.axis_index("s")
    pltpu.sync_copy(colloc_hbm.at[cid].at[sid], col_v)
    bufs = (rows0, rows1)
    sems = (sem0, sem1)
    for chunk in range(C):
        pltpu.sync_copy(zacc_hbm.at[pl.ds(sid * 328, 328)],
                        acc.at[pl.ds(sid * 328, 328)])
        pltpu.sync_copy(rowadj_hbm.at[chunk].at[sid], radj_v)
        plsc.subcore_barrier()
        cps = [None, None]
        cps[0] = pltpu.async_copy(xs_hbm.at[radj_v.at[0]], bufs[0], sems[0])
        for j in range(NBA):
            cur = j % 2
            nxt = 1 - cur
            if j + 1 < NBA:
                cps[nxt] = pltpu.async_copy(xs_hbm.at[radj_v.at[j + 1]],
                                            bufs[nxt], sems[nxt])
            cps[cur].wait()
            pltpu.sync_copy(bufs[cur], acc.at[col_v.at[j]], add=True)
        plsc.subcore_barrier()
        pltpu.sync_copy(
            acc.at[pl.ds(sid * 320, 320)],
            out_hbm.at[chunk].at[pl.ds(cid * HALF_N + sid * 320, 320)])
        plsc.subcore_barrier()


def _spmm(xs, rowadj, colloc, zacc, C):
    f = pl.kernel(
        functools.partial(_spmm_body, C),
        out_type=jax.ShapeDtypeStruct((C, 2 * HALF_N, 128), jnp.float32),
        mesh=_SC_MESH,
        scratch_types=[
            pltpu.VMEM((NBA, 128), jnp.int32),
            pltpu.VMEM((NBA, 128), jnp.int32),
            pltpu.VMEM((128, 128), jnp.float32),
            pltpu.VMEM((128, 128), jnp.float32),
            pltpu.SemaphoreType.DMA,
            pltpu.SemaphoreType.DMA,
            pltpu.VMEM_SHARED((ACC_ROWS, 128), jnp.float32),
        ],
    )
    return f(xs, rowadj, colloc, zacc)


def _decode_body(u_hbm, v_hbm, rp_hbm, cp_hbm, w2_hbm, b2_hbm, out_hbm,
                 rp_v, cp_v, ub, vb, w2_v, b2_v, out_v, semu, semv):
    cid = lax.axis_index("c")
    sid = lax.axis_index("s")
    wid = cid * 16 + sid
    pltpu.sync_copy(rp_hbm.at[wid], rp_v)
    pltpu.sync_copy(cp_hbm.at[wid], cp_v)
    pltpu.sync_copy(w2_hbm, w2_v)
    pltpu.sync_copy(b2_hbm, b2_v)
    w2s = [w2_v[k] for k in range(HID_CH // 16)]
    it = _iota16()

    def batch(jb, _):
        cu = pltpu.async_copy(u_hbm.at[rp_v.at[jb]], ub, semu)
        cv = pltpu.async_copy(v_hbm.at[cp_v.at[jb]], vb, semv)
        cu.wait()
        cv.wait()

        def edge(b, vec):
            acc = jnp.zeros((16,), jnp.float32)
            for k in range(HID_CH // 16):
                s = pl.ds(k * 16, 16)
                acc = acc + jnp.maximum(ub[b, s] + vb[b, s], 0.0) * w2s[k]
            for sh in (8, 4, 2, 1):
                acc = acc + _lane_shuffle(acc, it ^ sh)
            vec = jnp.where(it == (b & 15), acc, vec)

            @pl.when((b & 15) == 15)
            def _():
                out_v[pl.ds(jb * PB + (b // 16) * 16, 16)] = vec

            return vec

        lax.fori_loop(0, PB, edge, jnp.zeros((16,), jnp.float32))
        return 0

    lax.fori_loop(0, NPB, batch, 0)

    b2s = b2_v[...]

    def sig(i, _):
        s = pl.ds(i * 16, 16)
        xx = out_v[s] + b2s
        out_v[s] = 1.0 / (1.0 + jnp.exp(-xx))
        return 0

    lax.fori_loop(0, PPW // 16, sig, 0)
    pltpu.sync_copy(out_v, out_hbm.at[pl.ds(wid * PPW, PPW)])


def _decode(u, v, rp_sc, cp_sc, w2, b2):
    f = pl.kernel(
        _decode_body,
        out_type=jax.ShapeDtypeStruct((32 * PPW,), jnp.float32),
        mesh=_SC_MESH,
        scratch_types=[
            pltpu.VMEM((NPB, PB), jnp.int32),
            pltpu.VMEM((NPB, PB), jnp.int32),
            pltpu.VMEM((PB, HID_CH), jnp.float32),
            pltpu.VMEM((PB, HID_CH), jnp.float32),
            pltpu.VMEM((HID_CH // 16, 16), jnp.float32),
            pltpu.VMEM((16,), jnp.float32),
            pltpu.VMEM((PPW,), jnp.float32),
            pltpu.SemaphoreType.DMA,
            pltpu.SemaphoreType.DMA,
        ],
    )
    return f(u, v, rp_sc, cp_sc, w2, b2)


_RB = 1000


def _mm1_body(x_ref, w_ref, h_ref, xs_ref, dis_ref):
    deg = h_ref[0, :, 0:1] + h_ref[1, :, 0:1]
    dis = jax.lax.rsqrt(deg)
    y = jnp.dot(x_ref[...], w_ref[...], preferred_element_type=jnp.float32)
    xs_ref[0] = y * dis
    dis_ref[...] = dis


def _mm1(x, W1, hist):
    nc = HID_CH // 128
    return pl.pallas_call(
        _mm1_body,
        grid=(N_NODES // _RB, nc),
        in_specs=[
            pl.BlockSpec((_RB, IN_CH), lambda i, c: (i, 0)),
            pl.BlockSpec((IN_CH, 128), lambda i, c: (0, c)),
            pl.BlockSpec((2, _RB, 16), lambda i, c: (0, i, 0)),
        ],
        out_specs=[
            pl.BlockSpec((1, _RB, 128), lambda i, c: (c, i, 0)),
            pl.BlockSpec((_RB, 1), lambda i, c: (i, 0)),
        ],
        out_shape=[
            jax.ShapeDtypeStruct((nc, N_NODES, 128), jnp.float32),
            jax.ShapeDtypeStruct((N_NODES, 1), jnp.float32),
        ],
    )(x, W1, hist)


def _mm2_body(acc_ref, w_ref, b_ref, dis_ref, xs_ref):
    dis = dis_ref[...]
    y = jnp.zeros((_RB, 128), jnp.float32)
    for k in range(HID_CH // 128):
        zk = jnp.maximum(acc_ref[k] * dis + b_ref[0, k * 128:(k + 1) * 128],
                         0.0)
        y = y + jnp.dot(zk, w_ref[k * 128:(k + 1) * 128, :],
                        preferred_element_type=jnp.float32)
    xs_ref[0] = y * dis


def _mm2(acc1, W2, b1, dis):
    nc = OUT_CH // 128
    return pl.pallas_call(
        _mm2_body,
        grid=(N_NODES // _RB, nc),
        in_specs=[
            pl.BlockSpec((HID_CH // 128, _RB, 128), lambda i, c: (0, i, 0)),
            pl.BlockSpec((HID_CH, 128), lambda i, c: (0, c)),
            pl.BlockSpec((1, HID_CH), lambda i, c: (0, 0)),
            pl.BlockSpec((_RB, 1), lambda i, c: (i, 0)),
        ],
        out_specs=pl.BlockSpec((1, _RB, 128), lambda i, c: (c, i, 0)),
        out_shape=jax.ShapeDtypeStruct((nc, N_NODES, 128), jnp.float32),
    )(acc1, W2, b1, dis)


def _mm3_body(acc_ref, wa_ref, wb_ref, b2_ref, blp_ref, dis_ref,
              u_ref, v_ref):
    dis = dis_ref[...]
    parts = [acc_ref[k] * dis + b2_ref[0, k * 128:(k + 1) * 128]
             for k in range(OUT_CH // 128)]
    z2 = jnp.concatenate(parts, axis=1)
    u_ref[...] = jnp.dot(z2, wa_ref[...],
                         preferred_element_type=jnp.float32) + blp_ref[...]
    v_ref[...] = jnp.dot(z2, wb_ref[...], preferred_element_type=jnp.float32)


def _mm3(acc2, W1a, W1b, b2, b1lp, dis):
    return pl.pallas_call(
        _mm3_body,
        grid=(N_NODES // _RB,),
        in_specs=[
            pl.BlockSpec((OUT_CH // 128, _RB, 128), lambda i: (0, i, 0)),
            pl.BlockSpec((OUT_CH, HID_CH), lambda i: (0, 0)),
            pl.BlockSpec((OUT_CH, HID_CH), lambda i: (0, 0)),
            pl.BlockSpec((1, OUT_CH), lambda i: (0, 0)),
            pl.BlockSpec((1, HID_CH), lambda i: (0, 0)),
            pl.BlockSpec((_RB, 1), lambda i: (i, 0)),
        ],
        out_specs=[
            pl.BlockSpec((_RB, HID_CH), lambda i: (i, 0)),
            pl.BlockSpec((_RB, HID_CH), lambda i: (i, 0)),
        ],
        out_shape=[
            jax.ShapeDtypeStruct((N_NODES, HID_CH), jnp.float32),
            jax.ShapeDtypeStruct((N_NODES, HID_CH), jnp.float32),
        ],
    )(acc2, W1a, W1b, b2, b1lp, dis)


def kernel(x, edge_index, pred_edge_index, conv1_W, conv1_b, conv2_W, conv2_b,
           lp_W1, lp_b1, lp_W2, lp_b2):
    loop = jnp.arange(N_NODES, dtype=jnp.int32)
    npad = E_TOT - E_REAL
    row = jnp.concatenate([edge_index[0].astype(jnp.int32), loop,
                           jnp.zeros((npad,), jnp.int32)])
    col = jnp.concatenate([edge_index[1].astype(jnp.int32), loop,
                           jnp.full((npad,), PAD_COL, jnp.int32)])
    col32 = col.reshape(32, E_TOT // 32 // 128, 128)
    nc1 = HID_CH // 128
    rowadj = (row.reshape(1, 16, NBA, 128)
              + (jnp.arange(nc1, dtype=jnp.int32) * N_NODES)[:, None, None,
                                                             None])
    base = (jnp.arange(2, dtype=jnp.int32) * HALF_N)[:, None, None, None]
    colg = col.reshape(1, 16, NBA, 128)
    inhalf = (colg >= base) & (colg < base + HALF_N)
    colloc = jnp.where(inhalf, colg - base, TRASH_L)
    ones16 = jnp.ones((128, 16), jnp.float32)
    zeros16 = jnp.zeros((HIST_ROWS, 16), jnp.float32)
    zacc = jnp.zeros((ACC_ROWS, 128), jnp.float32)

    hist = _deg(col32, ones16, zeros16)

    xs1, dis = _mm1(x, conv1_W, hist)
    acc1 = _spmm(xs1.reshape(-1, 128), rowadj, colloc, zacc, nc1)
    xs2 = _mm2(acc1[:, :N_NODES], conv2_W, conv1_b.reshape(1, HID_CH), dis)
    acc2 = _spmm(xs2.reshape(-1, 128), rowadj[:OUT_CH // 128], colloc, zacc,
                 OUT_CH // 128)
    u, v = _mm3(acc2[:, :N_NODES], lp_W1[:OUT_CH], lp_W1[OUT_CH:],
                conv2_b.reshape(1, OUT_CH), lp_b1.reshape(1, HID_CH), dis)

    ppad = P_TOT - P_RAW
    rp = jnp.concatenate([pred_edge_index[0].astype(jnp.int32),
                          jnp.zeros((ppad,), jnp.int32)]).reshape(32, NPB, PB)
    cp = jnp.concatenate([pred_edge_index[1].astype(jnp.int32),
                          jnp.zeros((ppad,), jnp.int32)]).reshape(32, NPB, PB)
    w2 = lp_W2.reshape(HID_CH // 16, 16)
    b2 = jnp.broadcast_to(lp_b2, (16,))
    logits = _decode(u, v, rp, cp, w2, b2)
    return logits.reshape(P_TOT, 1)[:P_RAW]


# speedup vs baseline: 3.9594x; 1.0832x over previous
import functools

import jax
import jax.numpy as jnp
from jax import lax
from jax.experimental import pallas as pl
from jax.experimental.pallas import tpu as pltpu
from jax.experimental.pallas import tpu_sc as plsc

N_NODES = 10000
IN_CH = 256
HID_CH = 512
OUT_CH = 256

E_RAW = 160000
E_REAL = E_RAW + N_NODES
E_TOT = 172032
NBA = 84
HALF_N = 5120
ACC_ROWS = 5248
TRASH_L = 5120
PAD_COL = 10240
HIST_ROWS = 10368

P_RAW = 100000
P_TOT = 100352
PPW = 3136
PB = 32
NPB = PPW // PB

_SC_MESH = plsc.VectorSubcoreMesh(core_axis_name="c", subcore_axis_name="s")


def _iota16():
    return lax.broadcasted_iota(jnp.int32, (16,), 0)


def _lane_shuffle(x, idx):
    dn = lax.GatherDimensionNumbers(offset_dims=(), collapsed_slice_dims=(0,),
                                    start_index_map=(0,))
    return lax.gather(x, idx[:, None], dn, slice_sizes=(1,),
                      mode=lax.GatherScatterMode.PROMISE_IN_BOUNDS)


def _deg_body(col_hbm, ones_hbm, zeros_hbm, hist_hbm, col_v, ones_v, hist_s):
    cid = lax.axis_index("c")
    sid = lax.axis_index("s")
    wid = cid * 16 + sid
    pltpu.sync_copy(col_hbm.at[wid], col_v)
    pltpu.sync_copy(ones_hbm, ones_v)
    pltpu.sync_copy(zeros_hbm.at[pl.ds(sid * 648, 648)],
                    hist_s.at[pl.ds(sid * 648, 648)])
    plsc.subcore_barrier()
    for j in range(E_TOT // 32 // 128):
        pltpu.sync_copy(ones_v, hist_s.at[col_v.at[j]], add=True)
    plsc.subcore_barrier()
    pltpu.sync_copy(hist_s.at[pl.ds(sid * 648, 648)],
                    hist_hbm.at[cid].at[pl.ds(sid * 648, 648)])


def _deg(col32, ones16, zeros16):
    f = pl.kernel(
        _deg_body,
        out_type=jax.ShapeDtypeStruct((2, HIST_ROWS, 16), jnp.float32),
        mesh=_SC_MESH,
        scratch_types=[
            pltpu.VMEM((E_TOT // 32 // 128, 128), jnp.int32),
            pltpu.VMEM((128, 16), jnp.float32),
            pltpu.VMEM_SHARED((HIST_ROWS, 16), jnp.float32),
        ],
    )
    return f(col32, ones16, zeros16)


def _spmm_body(C, xs_hbm, rowadj_hbm, colloc_hbm, zacc_hbm, out_hbm,
               radj_v, col_v, bufs, gsems, ssems, acc):
    cid = lax.axis_index("c")
    sid = lax.axis_index("s")
    pltpu.sync_copy(colloc_hbm.at[cid].at[sid], col_v)
    nbuf = len(bufs)
    for chunk in range(C):
        pltpu.sync_copy(zacc_hbm.at[pl.ds(sid * 328, 328)],
                        acc.at[pl.ds(sid * 328, 328)])
        pltpu.sync_copy(rowadj_hbm.at[chunk].at[sid], radj_v)
        plsc.subcore_barrier()
        gcps = [None] * NBA
        scps = [None] * NBA
        swaited = set()
        for j in range(min(4, NBA)):
            gcps[j] = pltpu.async_copy(xs_hbm.at[radj_v.at[j]],
                                       bufs[j % nbuf], gsems[j % nbuf])
        for j in range(NBA):
            gcps[j].wait()
            pltpu.sync_copy(bufs[j % nbuf], acc.at[col_v.at[j]], add=True)
            scps[j] = None
            swaited.add(j)
            if j + 4 < NBA:
                gcps[j + 4] = pltpu.async_copy(
                    xs_hbm.at[radj_v.at[j + 4]],
                    bufs[(j + 4) % nbuf], gsems[(j + 4) % nbuf])
        for j in range(NBA):
            if j not in swaited and scps[j] is not None:
                scps[j].wait()
        plsc.subcore_barrier()
        pltpu.sync_copy(
            acc.at[pl.ds(sid * 320, 320)],
            out_hbm.at[chunk].at[pl.ds(cid * HALF_N + sid * 320, 320)])
        plsc.subcore_barrier()


def _spmm(xs, rowadj, colloc, zacc, C):
    f = pl.kernel(
        functools.partial(_spmm_body, C),
        out_type=jax.ShapeDtypeStruct((C, 2 * HALF_N, 128), jnp.float32),
        mesh=_SC_MESH,
        scratch_types=[
            pltpu.VMEM((NBA, 128), jnp.int32),
            pltpu.VMEM((NBA, 128), jnp.int32),
            [pltpu.VMEM((128, 128), jnp.float32) for _ in range(4)],
            [pltpu.SemaphoreType.DMA for _ in range(4)],
            [pltpu.SemaphoreType.DMA for _ in range(4)],
            pltpu.VMEM_SHARED((ACC_ROWS, 128), jnp.float32),
        ],
    )
    return f(xs, rowadj, colloc, zacc)


def _decode_body(u_hbm, v_hbm, rp_hbm, cp_hbm, w2_hbm, b2_hbm, out_hbm,
                 rp_v, cp_v, ub0, ub1, vb0, vb1, w2_v, b2_v, out_v,
                 semu0, semu1, semv0, semv1):
    cid = lax.axis_index("c")
    sid = lax.axis_index("s")
    wid = cid * 16 + sid
    pltpu.sync_copy(rp_hbm.at[wid], rp_v)
    pltpu.sync_copy(cp_hbm.at[wid], cp_v)
    pltpu.sync_copy(w2_hbm, w2_v)
    pltpu.sync_copy(b2_hbm, b2_v)
    w2s = [w2_v[k] for k in range(HID_CH // 16)]
    it = _iota16()
    ubs = (ub0, ub1)
    vbs = (vb0, vb1)
    us = (semu0, semu1)
    vs = (semv0, semv1)

    def issue(jb, p):
        pltpu.async_copy(u_hbm.at[rp_v.at[jb]], ubs[p], us[p])
        pltpu.async_copy(v_hbm.at[cp_v.at[jb]], vbs[p], vs[p])

    def wait(jb, p):
        pltpu.make_async_copy(u_hbm.at[rp_v.at[jb]], ubs[p], us[p]).wait()
        pltpu.make_async_copy(v_hbm.at[cp_v.at[jb]], vbs[p], vs[p]).wait()

    def compute(jb, p):
        ubuf = ubs[p]
        vbuf = vbs[p]

        def edge(b, vec):
            acc = jnp.zeros((16,), jnp.float32)
            for k in range(HID_CH // 16):
                s = pl.ds(k * 16, 16)
                acc = acc + jnp.maximum(ubuf[b, s] + vbuf[b, s], 0.0) * w2s[k]
            for sh in (8, 4, 2, 1):
                acc = acc + _lane_shuffle(acc, it ^ sh)
            vec = jnp.where(it == (b & 15), acc, vec)

            @pl.when((b & 15) == 15)
            def _():
                out_v[pl.ds(jb * PB + (b // 16) * 16, 16)] = vec

            return vec

        lax.fori_loop(0, PB, edge, jnp.zeros((16,), jnp.float32))

    issue(0, 0)

    def pairstep(jp, _):
        j0 = jp * 2

        @pl.when(j0 + 1 < NPB)
        def _():
            issue(j0 + 1, 1)

        wait(j0, 0)
        compute(j0, 0)

        @pl.when(j0 + 2 < NPB)
        def _():
            issue(j0 + 2, 0)

        @pl.when(j0 + 1 < NPB)
        def _():
            wait(j0 + 1, 1)
            compute(j0 + 1, 1)

        return 0

    lax.fori_loop(0, (NPB + 1) // 2, pairstep, 0)

    b2s = b2_v[...]

    def sig(i, _):
        s = pl.ds(i * 16, 16)
        xx = out_v[s] + b2s
        out_v[s] = 1.0 / (1.0 + jnp.exp(-xx))
        return 0

    lax.fori_loop(0, PPW // 16, sig, 0)
    pltpu.sync_copy(out_v, out_hbm.at[pl.ds(wid * PPW, PPW)])


def _decode(u, v, rp_sc, cp_sc, w2, b2):
    f = pl.kernel(
        _decode_body,
        out_type=jax.ShapeDtypeStruct((32 * PPW,), jnp.float32),
        mesh=_SC_MESH,
        scratch_types=[
            pltpu.VMEM((NPB, PB), jnp.int32),
            pltpu.VMEM((NPB, PB), jnp.int32),
            pltpu.VMEM((PB, HID_CH), jnp.float32),
            pltpu.VMEM((PB, HID_CH), jnp.float32),
            pltpu.VMEM((PB, HID_CH), jnp.float32),
            pltpu.VMEM((PB, HID_CH), jnp.float32),
            pltpu.VMEM((HID_CH // 16, 16), jnp.float32),
            pltpu.VMEM((16,), jnp.float32),
            pltpu.VMEM((PPW,), jnp.float32),
            pltpu.SemaphoreType.DMA,
            pltpu.SemaphoreType.DMA,
            pltpu.SemaphoreType.DMA,
            pltpu.SemaphoreType.DMA,
        ],
    )
    return f(u, v, rp_sc, cp_sc, w2, b2)


_RB = 1000


def _mm1_body(x_ref, w_ref, h_ref, xs_ref, dis_ref):
    deg = h_ref[0, :, 0:1] + h_ref[1, :, 0:1]
    dis = jax.lax.rsqrt(deg)
    y = jnp.dot(x_ref[...], w_ref[...], preferred_element_type=jnp.float32)
    xs_ref[0] = y * dis
    dis_ref[...] = dis


def _mm1(x, W1, hist):
    nc = HID_CH // 128
    return pl.pallas_call(
        _mm1_body,
        grid=(N_NODES // _RB, nc),
        in_specs=[
            pl.BlockSpec((_RB, IN_CH), lambda i, c: (i, 0)),
            pl.BlockSpec((IN_CH, 128), lambda i, c: (0, c)),
            pl.BlockSpec((2, _RB, 16), lambda i, c: (0, i, 0)),
        ],
        out_specs=[
            pl.BlockSpec((1, _RB, 128), lambda i, c: (c, i, 0)),
            pl.BlockSpec((_RB, 1), lambda i, c: (i, 0)),
        ],
        out_shape=[
            jax.ShapeDtypeStruct((nc, N_NODES, 128), jnp.float32),
            jax.ShapeDtypeStruct((N_NODES, 1), jnp.float32),
        ],
    )(x, W1, hist)


def _mm2_body(acc_ref, w_ref, b_ref, dis_ref, xs_ref):
    dis = dis_ref[...]
    y = jnp.zeros((_RB, 128), jnp.float32)
    for k in range(HID_CH // 128):
        zk = jnp.maximum(acc_ref[k] * dis + b_ref[0, k * 128:(k + 1) * 128],
                         0.0)
        y = y + jnp.dot(zk, w_ref[k * 128:(k + 1) * 128, :],
                        preferred_element_type=jnp.float32)
    xs_ref[0] = y * dis


def _mm2(acc1, W2, b1, dis):
    nc = OUT_CH // 128
    return pl.pallas_call(
        _mm2_body,
        grid=(N_NODES // _RB, nc),
        in_specs=[
            pl.BlockSpec((HID_CH // 128, _RB, 128), lambda i, c: (0, i, 0)),
            pl.BlockSpec((HID_CH, 128), lambda i, c: (0, c)),
            pl.BlockSpec((1, HID_CH), lambda i, c: (0, 0)),
            pl.BlockSpec((_RB, 1), lambda i, c: (i, 0)),
        ],
        out_specs=pl.BlockSpec((1, _RB, 128), lambda i, c: (c, i, 0)),
        out_shape=jax.ShapeDtypeStruct((nc, N_NODES, 128), jnp.float32),
    )(acc1, W2, b1, dis)


def _mm3_body(acc_ref, wa_ref, wb_ref, b2_ref, blp_ref, dis_ref,
              u_ref, v_ref):
    dis = dis_ref[...]
    parts = [acc_ref[k] * dis + b2_ref[0, k * 128:(k + 1) * 128]
             for k in range(OUT_CH // 128)]
    z2 = jnp.concatenate(parts, axis=1)
    u_ref[...] = jnp.dot(z2, wa_ref[...],
                         preferred_element_type=jnp.float32) + blp_ref[...]
    v_ref[...] = jnp.dot(z2, wb_ref[...], preferred_element_type=jnp.float32)


def _mm3(acc2, W1a, W1b, b2, b1lp, dis):
    return pl.pallas_call(
        _mm3_body,
        grid=(N_NODES // _RB,),
        in_specs=[
            pl.BlockSpec((OUT_CH // 128, _RB, 128), lambda i: (0, i, 0)),
            pl.BlockSpec((OUT_CH, HID_CH), lambda i: (0, 0)),
            pl.BlockSpec((OUT_CH, HID_CH), lambda i: (0, 0)),
            pl.BlockSpec((1, OUT_CH), lambda i: (0, 0)),
            pl.BlockSpec((1, HID_CH), lambda i: (0, 0)),
            pl.BlockSpec((_RB, 1), lambda i: (i, 0)),
        ],
        out_specs=[
            pl.BlockSpec((_RB, HID_CH), lambda i: (i, 0)),
            pl.BlockSpec((_RB, HID_CH), lambda i: (i, 0)),
        ],
        out_shape=[
            jax.ShapeDtypeStruct((N_NODES, HID_CH), jnp.float32),
            jax.ShapeDtypeStruct((N_NODES, HID_CH), jnp.float32),
        ],
    )(acc2, W1a, W1b, b2, b1lp, dis)


def kernel(x, edge_index, pred_edge_index, conv1_W, conv1_b, conv2_W, conv2_b,
           lp_W1, lp_b1, lp_W2, lp_b2):
    loop = jnp.arange(N_NODES, dtype=jnp.int32)
    npad = E_TOT - E_REAL
    row = jnp.concatenate([edge_index[0].astype(jnp.int32), loop,
                           jnp.zeros((npad,), jnp.int32)])
    col = jnp.concatenate([edge_index[1].astype(jnp.int32), loop,
                           jnp.full((npad,), PAD_COL, jnp.int32)])
    col32 = col.reshape(32, E_TOT // 32 // 128, 128)
    nc1 = HID_CH // 128
    rowadj = (row.reshape(1, 16, NBA, 128)
              + (jnp.arange(nc1, dtype=jnp.int32) * N_NODES)[:, None, None,
                                                             None])
    base = (jnp.arange(2, dtype=jnp.int32) * HALF_N)[:, None, None, None]
    colg = col.reshape(1, 16, NBA, 128)
    inhalf = (colg >= base) & (colg < base + HALF_N)
    colloc = jnp.where(inhalf, colg - base, TRASH_L)
    ones16 = jnp.ones((128, 16), jnp.float32)
    zeros16 = jnp.zeros((HIST_ROWS, 16), jnp.float32)
    zacc = jnp.zeros((ACC_ROWS, 128), jnp.float32)

    hist = _deg(col32, ones16, zeros16)

    xs1, dis = _mm1(x, conv1_W, hist)
    acc1 = _spmm(xs1.reshape(-1, 128), rowadj, colloc, zacc, nc1)
    xs2 = _mm2(acc1[:, :N_NODES], conv2_W, conv1_b.reshape(1, HID_CH), dis)
    acc2 = _spmm(xs2.reshape(-1, 128), rowadj[:OUT_CH // 128], colloc, zacc,
                 OUT_CH // 128)
    u, v = _mm3(acc2[:, :N_NODES], lp_W1[:OUT_CH], lp_W1[OUT_CH:],
                conv2_b.reshape(1, OUT_CH), lp_b1.reshape(1, HID_CH), dis)

    ppad = P_TOT - P_RAW
    rp = jnp.concatenate([pred_edge_index[0].astype(jnp.int32),
                          jnp.zeros((ppad,), jnp.int32)]).reshape(32, NPB, PB)
    cp = jnp.concatenate([pred_edge_index[1].astype(jnp.int32),
                          jnp.zeros((ppad,), jnp.int32)]).reshape(32, NPB, PB)
    w2 = lp_W2.reshape(HID_CH // 16, 16)
    b2 = jnp.broadcast_to(lp_b2, (16,))
    logits = _decode(u, v, rp, cp, w2, b2)
    return logits.reshape(P_TOT, 1)[:P_RAW]


# speedup vs baseline: 3.9809x; 1.0054x over previous
import functools

import jax
import jax.numpy as jnp
from jax import lax
from jax.experimental import pallas as pl
from jax.experimental.pallas import tpu as pltpu
from jax.experimental.pallas import tpu_sc as plsc

N_NODES = 10000
IN_CH = 256
HID_CH = 512
OUT_CH = 256

E_RAW = 160000
E_REAL = E_RAW + N_NODES
E_TOT = 172032
NBA = 84
HALF_N = 5120
ACC_ROWS = 5248
TRASH_L = 5120
PAD_COL = 10240
HIST_ROWS = 10368

E_CAPB = 1344
WMAX = 88

P_RAW = 100000
P_TOT = 100352
PPW = 3136
PB = 32
NPB = PPW // PB

_SC_MESH = plsc.VectorSubcoreMesh(core_axis_name="c", subcore_axis_name="s")


def _iota16():
    return lax.broadcasted_iota(jnp.int32, (16,), 0)


def _lane_shuffle(x, idx):
    dn = lax.GatherDimensionNumbers(offset_dims=(), collapsed_slice_dims=(0,),
                                    start_index_map=(0,))
    return lax.gather(x, idx[:, None], dn, slice_sizes=(1,),
                      mode=lax.GatherScatterMode.PROMISE_IN_BOUNDS)


def _deg_body(col_hbm, ones_hbm, zeros_hbm, hist_hbm, col_v, ones_v, hist_s):
    cid = lax.axis_index("c")
    sid = lax.axis_index("s")
    wid = cid * 16 + sid
    pltpu.sync_copy(col_hbm.at[wid], col_v)
    pltpu.sync_copy(ones_hbm, ones_v)
    pltpu.sync_copy(zeros_hbm.at[pl.ds(sid * 648, 648)],
                    hist_s.at[pl.ds(sid * 648, 648)])
    plsc.subcore_barrier()
    for j in range(E_TOT // 32 // 128):
        pltpu.sync_copy(ones_v, hist_s.at[col_v.at[j]], add=True)
    plsc.subcore_barrier()
    pltpu.sync_copy(hist_s.at[pl.ds(sid * 648, 648)],
                    hist_hbm.at[cid].at[pl.ds(sid * 648, 648)])


def _deg(col32, ones16, zeros16):
    f = pl.kernel(
        _deg_body,
        out_type=jax.ShapeDtypeStruct((2, HIST_ROWS, 16), jnp.float32),
        mesh=_SC_MESH,
        scratch_types=[
            pltpu.VMEM((E_TOT // 32 // 128, 128), jnp.int32),
            pltpu.VMEM((128, 16), jnp.float32),
            pltpu.VMEM_SHARED((HIST_ROWS, 16), jnp.float32),
        ],
    )
    return f(col32, ones16, zeros16)


def _spmm_body(C, xs_hbm, rowadj_hbm, colloc_hbm, zacc_hbm, out_hbm,
               radj_v, col_v, bufs, gsems, acc):
    cid = lax.axis_index("c")
    sid = lax.axis_index("s")
    pltpu.sync_copy(colloc_hbm.at[cid * 16 + sid], col_v)

    def gissue(jj, p):
        return pltpu.async_copy(xs_hbm.at[radj_v.at[jj]], bufs[p], gsems[p])

    for chunk in range(C):
        pltpu.sync_copy(zacc_hbm.at[pl.ds(sid * 328, 328)],
                        acc.at[pl.ds(sid * 328, 328)])
        pltpu.sync_copy(rowadj_hbm.at[chunk * 16 + sid], radj_v)
        plsc.subcore_barrier()
        gcps = [None] * NBA
        for j in range(4):
            gcps[j] = gissue(j, j % 4)
        for j in range(NBA):
            gcps[j].wait()
            pltpu.sync_copy(bufs[j % 4], acc.at[col_v.at[j]], add=True)
            if j + 4 < NBA:
                gcps[j + 4] = gissue(j + 4, j % 4)
        plsc.subcore_barrier()
        pltpu.sync_copy(
            acc.at[pl.ds(sid * 320, 320)],
            out_hbm.at[chunk].at[pl.ds(cid * HALF_N + sid * 320, 320)])
        plsc.subcore_barrier()


def _spmm(xs, rowadj, colloc, zacc, C):
    f = pl.kernel(
        functools.partial(_spmm_body, C),
        out_type=jax.ShapeDtypeStruct((C, 2 * HALF_N, 128), jnp.float32),
        mesh=_SC_MESH,
        scratch_types=[
            pltpu.VMEM((NBA, 128), jnp.int32),
            pltpu.VMEM((NBA, 128), jnp.int32),
            [pltpu.VMEM((128, 128), jnp.float32) for _ in range(4)],
            [pltpu.SemaphoreType.DMA for _ in range(4)],
            pltpu.VMEM_SHARED((ACC_ROWS, 128), jnp.float32),
        ],
    )
    return f(xs, rowadj, colloc, zacc)


def _decode_body(u_hbm, v_hbm, rp_hbm, cp_hbm, w2_hbm, b2_hbm, out_hbm,
                 rp_v, cp_v, ub0, ub1, vb0, vb1, w2_v, b2_v, out_v,
                 semu0, semu1, semv0, semv1):
    cid = lax.axis_index("c")
    sid = lax.axis_index("s")
    wid = cid * 16 + sid
    pltpu.sync_copy(rp_hbm.at[wid], rp_v)
    pltpu.sync_copy(cp_hbm.at[wid], cp_v)
    pltpu.sync_copy(w2_hbm, w2_v)
    pltpu.sync_copy(b2_hbm, b2_v)
    w2s = [w2_v[k] for k in range(HID_CH // 16)]
    it = _iota16()
    ubs = (ub0, ub1)
    vbs = (vb0, vb1)
    us = (semu0, semu1)
    vs = (semv0, semv1)

    def issue(jb, p):
        pltpu.async_copy(u_hbm.at[rp_v.at[jb]], ubs[p], us[p])
        pltpu.async_copy(v_hbm.at[cp_v.at[jb]], vbs[p], vs[p])

    def wait(jb, p):
        pltpu.make_async_copy(u_hbm.at[rp_v.at[jb]], ubs[p], us[p]).wait()
        pltpu.make_async_copy(v_hbm.at[cp_v.at[jb]], vbs[p], vs[p]).wait()

    def compute(jb, p):
        ubuf = ubs[p]
        vbuf = vbs[p]

        def edge(b, vec):
            acc = jnp.zeros((16,), jnp.float32)
            for k in range(HID_CH // 16):
                s = pl.ds(k * 16, 16)
                acc = acc + jnp.maximum(ubuf[b, s] + vbuf[b, s], 0.0) * w2s[k]
            for sh in (8, 4, 2, 1):
                acc = acc + _lane_shuffle(acc, it ^ sh)
            vec = jnp.where(it == (b & 15), acc, vec)

            @pl.when((b & 15) == 15)
            def _():
                out_v[pl.ds(jb * PB + (b // 16) * 16, 16)] = vec

            return vec

        lax.fori_loop(0, PB, edge, jnp.zeros((16,), jnp.float32))

    issue(0, 0)

    def pairstep(jp, _):
        j0 = jp * 2

        @pl.when(j0 + 1 < NPB)
        def _():
            issue(j0 + 1, 1)

        wait(j0, 0)
        compute(j0, 0)

        @pl.when(j0 + 2 < NPB)
        def _():
            issue(j0 + 2, 0)

        @pl.when(j0 + 1 < NPB)
        def _():
            wait(j0 + 1, 1)
            compute(j0 + 1, 1)

        return 0

    lax.fori_loop(0, (NPB + 1) // 2, pairstep, 0)

    b2s = b2_v[...]

    def sig(i, _):
        s = pl.ds(i * 16, 16)
        xx = out_v[s] + b2s
        out_v[s] = 1.0 / (1.0 + jnp.exp(-xx))
        return 0

    lax.fori_loop(0, PPW // 16, sig, 0)
    pltpu.sync_copy(out_v, out_hbm.at[pl.ds(wid * PPW, PPW)])


def _decode(u, v, rp_sc, cp_sc, w2, b2):
    f = pl.kernel(
        _decode_body,
        out_type=jax.ShapeDtypeStruct((32 * PPW,), jnp.float32),
        mesh=_SC_MESH,
        scratch_types=[
            pltpu.VMEM((NPB, PB), jnp.int32),
            pltpu.VMEM((NPB, PB), jnp.int32),
            pltpu.VMEM((PB, HID_CH), jnp.float32),
            pltpu.VMEM((PB, HID_CH), jnp.float32),
            pltpu.VMEM((PB, HID_CH), jnp.float32),
            pltpu.VMEM((PB, HID_CH), jnp.float32),
            pltpu.VMEM((HID_CH // 16, 16), jnp.float32),
            pltpu.VMEM((16,), jnp.float32),
            pltpu.VMEM((PPW,), jnp.float32),
            pltpu.SemaphoreType.DMA,
            pltpu.SemaphoreType.DMA,
            pltpu.SemaphoreType.DMA,
            pltpu.SemaphoreType.DMA,
        ],
    )
    return f(u, v, rp_sc, cp_sc, w2, b2)


_RB = 1000


def _mm1_body(x_ref, w_ref, h_ref, xs_ref, dis_ref):
    deg = h_ref[0, :, 0:1] + h_ref[1, :, 0:1]
    dis = jax.lax.rsqrt(deg)
    y = jnp.dot(x_ref[...], w_ref[...], preferred_element_type=jnp.float32)
    xs_ref[0] = y * dis
    dis_ref[...] = dis


def _mm1(x, W1, hist):
    nc = HID_CH // 128
    return pl.pallas_call(
        _mm1_body,
        grid=(N_NODES // _RB, nc),
        in_specs=[
            pl.BlockSpec((_RB, IN_CH), lambda i, c: (i, 0)),
            pl.BlockSpec((IN_CH, 128), lambda i, c: (0, c)),
            pl.BlockSpec((2, _RB, 16), lambda i, c: (0, i, 0)),
        ],
        out_specs=[
            pl.BlockSpec((1, _RB, 128), lambda i, c: (c, i, 0)),
            pl.BlockSpec((_RB, 1), lambda i, c: (i, 0)),
        ],
        out_shape=[
            jax.ShapeDtypeStruct((nc, N_NODES, 128), jnp.float32),
            jax.ShapeDtypeStruct((N_NODES, 1), jnp.float32),
        ],
    )(x, W1, hist)


def _mm2_body(acc_ref, w_ref, b_ref, dis_ref, xs_ref):
    dis = dis_ref[...]
    y = jnp.zeros((_RB, 128), jnp.float32)
    for k in range(HID_CH // 128):
        zk = jnp.maximum(acc_ref[k] * dis + b_ref[0, k * 128:(k + 1) * 128],
                         0.0)
        y = y + jnp.dot(zk, w_ref[k * 128:(k + 1) * 128, :],
                        preferred_element_type=jnp.float32)
    xs_ref[0] = y * dis


def _mm2(acc1, W2, b1, dis):
    nc = OUT_CH // 128
    return pl.pallas_call(
        _mm2_body,
        grid=(N_NODES // _RB, nc),
        in_specs=[
            pl.BlockSpec((HID_CH // 128, _RB, 128), lambda i, c: (0, i, 0)),
            pl.BlockSpec((HID_CH, 128), lambda i, c: (0, c)),
            pl.BlockSpec((1, HID_CH), lambda i, c: (0, 0)),
            pl.BlockSpec((_RB, 1), lambda i, c: (i, 0)),
        ],
        out_specs=pl.BlockSpec((1, _RB, 128), lambda i, c: (c, i, 0)),
        out_shape=jax.ShapeDtypeStruct((nc, N_NODES, 128), jnp.float32),
    )(acc1, W2, b1, dis)


def _mm3_body(acc_ref, wa_ref, wb_ref, b2_ref, blp_ref, dis_ref,
              u_ref, v_ref):
    dis = dis_ref[...]
    parts = [acc_ref[k] * dis + b2_ref[0, k * 128:(k + 1) * 128]
             for k in range(OUT_CH // 128)]
    z2 = jnp.concatenate(parts, axis=1)
    u_ref[...] = jnp.dot(z2, wa_ref[...],
                         preferred_element_type=jnp.float32) + blp_ref[...]
    v_ref[...] = jnp.dot(z2, wb_ref[...], preferred_element_type=jnp.float32)


def _mm3(acc2, W1a, W1b, b2, b1lp, dis):
    return pl.pallas_call(
        _mm3_body,
        grid=(N_NODES // _RB,),
        in_specs=[
            pl.BlockSpec((OUT_CH // 128, _RB, 128), lambda i: (0, i, 0)),
            pl.BlockSpec((OUT_CH, HID_CH), lambda i: (0, 0)),
            pl.BlockSpec((OUT_CH, HID_CH), lambda i: (0, 0)),
            pl.BlockSpec((1, OUT_CH), lambda i: (0, 0)),
            pl.BlockSpec((1, HID_CH), lambda i: (0, 0)),
            pl.BlockSpec((_RB, 1), lambda i: (i, 0)),
        ],
        out_specs=[
            pl.BlockSpec((_RB, HID_CH), lambda i: (i, 0)),
            pl.BlockSpec((_RB, HID_CH), lambda i: (i, 0)),
        ],
        out_shape=[
            jax.ShapeDtypeStruct((N_NODES, HID_CH), jnp.float32),
            jax.ShapeDtypeStruct((N_NODES, HID_CH), jnp.float32),
        ],
    )(acc2, W1a, W1b, b2, b1lp, dis)


def kernel(x, edge_index, pred_edge_index, conv1_W, conv1_b, conv2_W, conv2_b,
           lp_W1, lp_b1, lp_W2, lp_b2):
    loop = jnp.arange(N_NODES, dtype=jnp.int32)
    npad = E_TOT - E_REAL
    row = jnp.concatenate([edge_index[0].astype(jnp.int32), loop,
                           jnp.zeros((npad,), jnp.int32)])
    col = jnp.concatenate([edge_index[1].astype(jnp.int32), loop,
                           jnp.full((npad,), PAD_COL, jnp.int32)])
    col32 = col.reshape(32, E_TOT // 32 // 128, 128)
    nc1 = HID_CH // 128

    rowadj = (row.reshape(1, 16, NBA, 128)
              + (jnp.arange(nc1, dtype=jnp.int32) * N_NODES)[:, None, None,
                                                             None]
              ).reshape(nc1 * 16, NBA, 128)
    base = (jnp.arange(2, dtype=jnp.int32) * HALF_N)[:, None, None, None]
    colg = col.reshape(1, 16, NBA, 128)
    inhalf = (colg >= base) & (colg < base + HALF_N)
    colloc = jnp.where(inhalf, colg - base, TRASH_L).reshape(2 * 16, NBA, 128)
    ones16 = jnp.ones((128, 16), jnp.float32)
    zeros16 = jnp.zeros((HIST_ROWS, 16), jnp.float32)
    zacc = jnp.zeros((ACC_ROWS, 128), jnp.float32)

    hist = _deg(col32, ones16, zeros16)

    xs1, dis = _mm1(x, conv1_W, hist)
    acc1 = _spmm(xs1.reshape(-1, 128), rowadj, colloc, zacc, nc1)
    xs2 = _mm2(acc1[:, :N_NODES], conv2_W, conv1_b.reshape(1, HID_CH), dis)
    acc2 = _spmm(xs2.reshape(-1, 128), rowadj[:(OUT_CH // 128) * 16], colloc,
                 zacc, OUT_CH // 128)
    u, v = _mm3(acc2[:, :N_NODES], lp_W1[:OUT_CH], lp_W1[OUT_CH:],
                conv2_b.reshape(1, OUT_CH), lp_b1.reshape(1, HID_CH), dis)

    ppad = P_TOT - P_RAW
    rp = jnp.concatenate([pred_edge_index[0].astype(jnp.int32),
                          jnp.zeros((ppad,), jnp.int32)]).reshape(32, NPB, PB)
    cp = jnp.concatenate([pred_edge_index[1].astype(jnp.int32),
                          jnp.zeros((ppad,), jnp.int32)]).reshape(32, NPB, PB)
    w2 = lp_W2.reshape(HID_CH // 16, 16)
    b2 = jnp.broadcast_to(lp_b2, (16,))
    logits = _decode(u, v, rp, cp, w2, b2)
    return logits.reshape(P_TOT, 1)[:P_RAW]


# speedup vs baseline: 4.0190x; 1.0096x over previous
import functools

import jax
import jax.numpy as jnp
from jax import lax
from jax.experimental import pallas as pl
from jax.experimental.pallas import tpu as pltpu
from jax.experimental.pallas import tpu_sc as plsc

N_NODES = 10000
IN_CH = 256
HID_CH = 512
OUT_CH = 256

E_RAW = 160000
E_REAL = E_RAW + N_NODES
E_TOT = 172032
NBA = 84
HALF_N = 5120
ACC_ROWS = 5248
TRASH_L = 5120
PAD_COL = 10240
HIST_ROWS = 10368

E_CAPB = 1344
WMAX = 88

P_RAW = 100000
P_TOT = 100352
PPW = 3136
PB = 32
NPB = PPW // PB

_SC_MESH = plsc.VectorSubcoreMesh(core_axis_name="c", subcore_axis_name="s")


def _iota16():
    return lax.broadcasted_iota(jnp.int32, (16,), 0)


def _lane_shuffle(x, idx):
    dn = lax.GatherDimensionNumbers(offset_dims=(), collapsed_slice_dims=(0,),
                                    start_index_map=(0,))
    return lax.gather(x, idx[:, None], dn, slice_sizes=(1,),
                      mode=lax.GatherScatterMode.PROMISE_IN_BOUNDS)


def _deg_body(col_hbm, ones_hbm, zeros_hbm, hist_hbm, col_v, ones_v, hist_s):
    cid = lax.axis_index("c")
    sid = lax.axis_index("s")
    wid = cid * 16 + sid
    pltpu.sync_copy(col_hbm.at[wid], col_v)
    pltpu.sync_copy(ones_hbm, ones_v)
    pltpu.sync_copy(zeros_hbm.at[pl.ds(sid * 648, 648)],
                    hist_s.at[pl.ds(sid * 648, 648)])
    plsc.subcore_barrier()
    for j in range(E_TOT // 32 // 128):
        pltpu.sync_copy(ones_v, hist_s.at[col_v.at[j]], add=True)
    plsc.subcore_barrier()
    pltpu.sync_copy(hist_s.at[pl.ds(sid * 648, 648)],
                    hist_hbm.at[cid].at[pl.ds(sid * 648, 648)])


def _deg(col32, ones16, zeros16):
    f = pl.kernel(
        _deg_body,
        out_type=jax.ShapeDtypeStruct((2, HIST_ROWS, 16), jnp.float32),
        mesh=_SC_MESH,
        scratch_types=[
            pltpu.VMEM((E_TOT // 32 // 128, 128), jnp.int32),
            pltpu.VMEM((128, 16), jnp.float32),
            pltpu.VMEM_SHARED((HIST_ROWS, 16), jnp.float32),
        ],
    )
    return f(col32, ones16, zeros16)


def _spmm_body(C, xs_hbm, rowadj_hbm, colloc_hbm, zacc_hbm, out_hbm,
               radj_v, col_v, bufs, gsems, acc):
    cid = lax.axis_index("c")
    sid = lax.axis_index("s")
    pltpu.sync_copy(colloc_hbm.at[cid * 16 + sid], col_v)

    def gissue(jj, p):
        return pltpu.async_copy(xs_hbm.at[radj_v.at[jj]], bufs[p], gsems[p])

    for chunk in range(C):
        pltpu.sync_copy(zacc_hbm.at[pl.ds(sid * 328, 328)],
                        acc.at[pl.ds(sid * 328, 328)])
        pltpu.sync_copy(rowadj_hbm.at[chunk * 16 + sid], radj_v)
        plsc.subcore_barrier()
        gcps = [None] * NBA
        for j in range(4):
            gcps[j] = gissue(j, j % 4)
        for j in range(NBA):
            gcps[j].wait()
            pltpu.sync_copy(bufs[j % 4], acc.at[col_v.at[j]], add=True)
            if j + 4 < NBA:
                gcps[j + 4] = gissue(j + 4, j % 4)
        plsc.subcore_barrier()
        pltpu.sync_copy(
            acc.at[pl.ds(sid * 320, 320)],
            out_hbm.at[chunk].at[pl.ds(cid * HALF_N + sid * 320, 320)])
        plsc.subcore_barrier()


def _spmm(xs, rowadj, colloc, zacc, C):
    f = pl.kernel(
        functools.partial(_spmm_body, C),
        out_type=jax.ShapeDtypeStruct((C, 2 * HALF_N, 128), jnp.float32),
        mesh=_SC_MESH,
        scratch_types=[
            pltpu.VMEM((NBA, 128), jnp.int32),
            pltpu.VMEM((NBA, 128), jnp.int32),
            [pltpu.VMEM((128, 128), jnp.float32) for _ in range(4)],
            [pltpu.SemaphoreType.DMA for _ in range(4)],
            pltpu.VMEM_SHARED((ACC_ROWS, 128), jnp.float32),
        ],
    )
    return f(xs, rowadj, colloc, zacc)


def _decode_body(u_hbm, v_hbm, rp_hbm, cp_hbm, w2_hbm, b2_hbm, out_hbm,
                 rp_v, cp_v, ub0, ub1, vb0, vb1, w2_v, b2_v, out_v,
                 semu0, semu1, semv0, semv1):
    cid = lax.axis_index("c")
    sid = lax.axis_index("s")
    wid = cid * 16 + sid
    pltpu.sync_copy(rp_hbm.at[wid], rp_v)
    pltpu.sync_copy(cp_hbm.at[wid], cp_v)
    pltpu.sync_copy(w2_hbm, w2_v)
    pltpu.sync_copy(b2_hbm, b2_v)
    w2s = [w2_v[k] for k in range(HID_CH // 16)]
    it = _iota16()
    ubs = (ub0, ub1)
    vbs = (vb0, vb1)
    us = (semu0, semu1)
    vs = (semv0, semv1)

    def issue(jb, p):
        pltpu.async_copy(u_hbm.at[rp_v.at[jb]], ubs[p], us[p])
        pltpu.async_copy(v_hbm.at[cp_v.at[jb]], vbs[p], vs[p])

    def wait(jb, p):
        pltpu.make_async_copy(u_hbm.at[rp_v.at[jb]], ubs[p], us[p]).wait()
        pltpu.make_async_copy(v_hbm.at[cp_v.at[jb]], vbs[p], vs[p]).wait()

    def compute(jb, p):
        ubuf = ubs[p]
        vbuf = vbs[p]

        def edge(b, vec):
            acc = jnp.zeros((16,), jnp.float32)
            for k in range(HID_CH // 16):
                s = pl.ds(k * 16, 16)
                acc = acc + jnp.maximum(ubuf[b, s] + vbuf[b, s], 0.0) * w2s[k]
            for sh in (8, 4, 2, 1):
                acc = acc + _lane_shuffle(acc, it ^ sh)
            vec = jnp.where(it == (b & 15), acc, vec)

            @pl.when((b & 15) == 15)
            def _():
                out_v[pl.ds(jb * PB + (b // 16) * 16, 16)] = vec

            return vec

        lax.fori_loop(0, PB, edge, jnp.zeros((16,), jnp.float32))

    issue(0, 0)

    def pairstep(jp, _):
        j0 = jp * 2

        @pl.when(j0 + 1 < NPB)
        def _():
            issue(j0 + 1, 1)

        wait(j0, 0)
        compute(j0, 0)

        @pl.when(j0 + 2 < NPB)
        def _():
            issue(j0 + 2, 0)

        @pl.when(j0 + 1 < NPB)
        def _():
            wait(j0 + 1, 1)
            compute(j0 + 1, 1)

        return 0

    lax.fori_loop(0, (NPB + 1) // 2, pairstep, 0)

    b2s = b2_v[...]

    def sig(i, _):
        s = pl.ds(i * 16, 16)
        xx = out_v[s] + b2s
        out_v[s] = 1.0 / (1.0 + jnp.exp(-xx))
        return 0

    lax.fori_loop(0, PPW // 16, sig, 0)
    pltpu.sync_copy(out_v, out_hbm.at[pl.ds(wid * PPW, PPW)])


def _decode(u, v, rp_sc, cp_sc, w2, b2):
    f = pl.kernel(
        _decode_body,
        out_type=jax.ShapeDtypeStruct((32 * PPW,), jnp.float32),
        mesh=_SC_MESH,
        scratch_types=[
            pltpu.VMEM((NPB, PB), jnp.int32),
            pltpu.VMEM((NPB, PB), jnp.int32),
            pltpu.VMEM((PB, HID_CH), jnp.float32),
            pltpu.VMEM((PB, HID_CH), jnp.float32),
            pltpu.VMEM((PB, HID_CH), jnp.float32),
            pltpu.VMEM((PB, HID_CH), jnp.float32),
            pltpu.VMEM((HID_CH // 16, 16), jnp.float32),
            pltpu.VMEM((16,), jnp.float32),
            pltpu.VMEM((PPW,), jnp.float32),
            pltpu.SemaphoreType.DMA,
            pltpu.SemaphoreType.DMA,
            pltpu.SemaphoreType.DMA,
            pltpu.SemaphoreType.DMA,
        ],
    )
    return f(u, v, rp_sc, cp_sc, w2, b2)


_RB = 1000


def _mm1_body(x_ref, w_ref, h_ref, xs_ref, dis_ref):
    deg = h_ref[0, :, 0:1] + h_ref[1, :, 0:1]
    dis = jax.lax.rsqrt(deg)
    y = jnp.dot(x_ref[...], w_ref[...], preferred_element_type=jnp.float32)
    xs_ref[0] = y * dis
    dis_ref[...] = dis


def _mm1(x, W1, hist):
    nc = HID_CH // 128
    return pl.pallas_call(
        _mm1_body,
        grid=(N_NODES // _RB, nc),
        in_specs=[
            pl.BlockSpec((_RB, IN_CH), lambda i, c: (i, 0)),
            pl.BlockSpec((IN_CH, 128), lambda i, c: (0, c)),
            pl.BlockSpec((2, _RB, 16), lambda i, c: (0, i, 0)),
        ],
        out_specs=[
            pl.BlockSpec((1, _RB, 128), lambda i, c: (c, i, 0)),
            pl.BlockSpec((_RB, 1), lambda i, c: (i, 0)),
        ],
        out_shape=[
            jax.ShapeDtypeStruct((nc, N_NODES, 128), jnp.float32),
            jax.ShapeDtypeStruct((N_NODES, 1), jnp.float32),
        ],
    )(x, W1, hist)


def _mm2_body(acc_ref, w_ref, b_ref, dis_ref, xs_ref):
    dis = dis_ref[...]
    y = jnp.zeros((_RB, 128), jnp.float32)
    for k in range(HID_CH // 128):
        zk = jnp.maximum(acc_ref[k] * dis + b_ref[0, k * 128:(k + 1) * 128],
                         0.0)
        y = y + jnp.dot(zk, w_ref[k * 128:(k + 1) * 128, :],
                        preferred_element_type=jnp.float32)
    xs_ref[0] = y * dis


def _mm2(acc1, W2, b1, dis):
    nc = OUT_CH // 128
    return pl.pallas_call(
        _mm2_body,
        grid=(N_NODES // _RB, nc),
        in_specs=[
            pl.BlockSpec((HID_CH // 128, _RB, 128), lambda i, c: (0, i, 0)),
            pl.BlockSpec((HID_CH, 128), lambda i, c: (0, c)),
            pl.BlockSpec((1, HID_CH), lambda i, c: (0, 0)),
            pl.BlockSpec((_RB, 1), lambda i, c: (i, 0)),
        ],
        out_specs=pl.BlockSpec((1, _RB, 128), lambda i, c: (c, i, 0)),
        out_shape=jax.ShapeDtypeStruct((nc, N_NODES, 128), jnp.float32),
    )(acc1, W2, b1, dis)


def _mm3_body(acc_ref, wa_ref, wb_ref, b2_ref, blp_ref, dis_ref,
              u_ref, v_ref):
    dis = dis_ref[...]
    parts = [acc_ref[k] * dis + b2_ref[0, k * 128:(k + 1) * 128]
             for k in range(OUT_CH // 128)]
    z2 = jnp.concatenate(parts, axis=1)
    u_ref[...] = jnp.dot(z2, wa_ref[...],
                         preferred_element_type=jnp.float32) + blp_ref[...]
    v_ref[...] = jnp.dot(z2, wb_ref[...], preferred_element_type=jnp.float32)


def _mm3(acc2, W1a, W1b, b2, b1lp, dis):
    return pl.pallas_call(
        _mm3_body,
        grid=(N_NODES // _RB,),
        in_specs=[
            pl.BlockSpec((OUT_CH // 128, _RB, 128), lambda i: (0, i, 0)),
            pl.BlockSpec((OUT_CH, HID_CH), lambda i: (0, 0)),
            pl.BlockSpec((OUT_CH, HID_CH), lambda i: (0, 0)),
            pl.BlockSpec((1, OUT_CH), lambda i: (0, 0)),
            pl.BlockSpec((1, HID_CH), lambda i: (0, 0)),
            pl.BlockSpec((_RB, 1), lambda i: (i, 0)),
        ],
        out_specs=[
            pl.BlockSpec((_RB, HID_CH), lambda i: (i, 0)),
            pl.BlockSpec((_RB, HID_CH), lambda i: (i, 0)),
        ],
        out_shape=[
            jax.ShapeDtypeStruct((N_NODES, HID_CH), jnp.float32),
            jax.ShapeDtypeStruct((N_NODES, HID_CH), jnp.float32),
        ],
    )(acc2, W1a, W1b, b2, b1lp, dis)


def kernel(x, edge_index, pred_edge_index, conv1_W, conv1_b, conv2_W, conv2_b,
           lp_W1, lp_b1, lp_W2, lp_b2):
    loop = jnp.arange(N_NODES, dtype=jnp.int32)
    npad = E_TOT - E_REAL
    row = jnp.concatenate([edge_index[0].astype(jnp.int32), loop,
                           jnp.zeros((npad,), jnp.int32)])
    col = jnp.concatenate([edge_index[1].astype(jnp.int32), loop,
                           jnp.full((npad,), PAD_COL, jnp.int32)])
    col32 = col.reshape(32, E_TOT // 32 // 128, 128)
    nc1 = HID_CH // 128

    rowadj = (row.reshape(1, 16, NBA, 128)
              + (jnp.arange(nc1, dtype=jnp.int32) * N_NODES)[:, None, None,
                                                             None]
              ).reshape(nc1 * 16, NBA, 128)
    base = (jnp.arange(2, dtype=jnp.int32) * HALF_N)[:, None, None, None]
    colg = col.reshape(1, 16, NBA, 128)
    inhalf = (colg >= base) & (colg < base + HALF_N)
    colloc = jnp.where(inhalf, colg - base, TRASH_L).reshape(2 * 16, NBA, 128)
    ones16 = jnp.ones((128, 16), jnp.float32)
    zeros16 = jnp.zeros((HIST_ROWS, 16), jnp.float32)
    zacc = jnp.zeros((ACC_ROWS, 128), jnp.float32)

    hist = _deg(col32, ones16, zeros16)

    xs1, dis = _mm1(x, conv1_W, hist)
    acc1 = _spmm(xs1.reshape(-1, 128), rowadj, colloc, zacc, nc1)
    xs2 = _mm2(acc1, conv2_W, conv1_b.reshape(1, HID_CH), dis)
    acc2 = _spmm(xs2.reshape(-1, 128), rowadj[:(OUT_CH // 128) * 16], colloc,
                 zacc, OUT_CH // 128)
    u, v = _mm3(acc2, lp_W1[:OUT_CH], lp_W1[OUT_CH:],
                conv2_b.reshape(1, OUT_CH), lp_b1.reshape(1, HID_CH), dis)

    ppad = P_TOT - P_RAW
    rp = jnp.concatenate([pred_edge_index[0].astype(jnp.int32),
                          jnp.zeros((ppad,), jnp.int32)]).reshape(32, NPB, PB)
    cp = jnp.concatenate([pred_edge_index[1].astype(jnp.int32),
                          jnp.zeros((ppad,), jnp.int32)]).reshape(32, NPB, PB)
    w2 = lp_W2.reshape(HID_CH // 16, 16)
    b2 = jnp.broadcast_to(lp_b2, (16,))
    logits = _decode(u, v, rp, cp, w2, b2)
    return logits.reshape(P_TOT, 1)[:P_RAW]


# speedup vs baseline: 4.0193x; 1.0001x over previous
import functools

import jax
import jax.numpy as jnp
from jax import lax
from jax.experimental import pallas as pl
from jax.experimental.pallas import tpu as pltpu
from jax.experimental.pallas import tpu_sc as plsc

N_NODES = 10000
IN_CH = 256
HID_CH = 512
OUT_CH = 256

E_RAW = 160000
E_REAL = E_RAW + N_NODES
E_TOT = 172032
NBA = 84
HALF_N = 5120
ACC_ROWS = 5248
TRASH_L = 5120
PAD_COL = 10240
HIST_ROWS = 10368

P_RAW = 100000
P_TOT = 100352
PPW = 3136
PB = 32
NPB = PPW // PB

_SC_MESH = plsc.VectorSubcoreMesh(core_axis_name="c", subcore_axis_name="s")


def _iota16():
    return lax.broadcasted_iota(jnp.int32, (16,), 0)


def _lane_shuffle(x, idx):
    dn = lax.GatherDimensionNumbers(offset_dims=(), collapsed_slice_dims=(0,),
                                    start_index_map=(0,))
    return lax.gather(x, idx[:, None], dn, slice_sizes=(1,),
                      mode=lax.GatherScatterMode.PROMISE_IN_BOUNDS)


def _deg_body(col_hbm, ones_hbm, zeros_hbm, hist_hbm, col_v, ones_v, hist_s):
    cid = lax.axis_index("c")
    sid = lax.axis_index("s")
    wid = cid * 16 + sid
    pltpu.sync_copy(col_hbm.at[wid], col_v)
    pltpu.sync_copy(ones_hbm, ones_v)
    pltpu.sync_copy(zeros_hbm.at[pl.ds(sid * 648, 648)],
                    hist_s.at[pl.ds(sid * 648, 648)])
    plsc.subcore_barrier()
    for j in range(E_TOT // 32 // 128):
        pltpu.sync_copy(ones_v, hist_s.at[col_v.at[j]], add=True)
    plsc.subcore_barrier()
    pltpu.sync_copy(hist_s.at[pl.ds(sid * 648, 648)],
                    hist_hbm.at[cid].at[pl.ds(sid * 648, 648)])


def _deg(col32, ones16, zeros16):
    f = pl.kernel(
        _deg_body,
        out_type=jax.ShapeDtypeStruct((2, HIST_ROWS, 16), jnp.float32),
        mesh=_SC_MESH,
        scratch_types=[
            pltpu.VMEM((E_TOT // 32 // 128, 128), jnp.int32),
            pltpu.VMEM((128, 16), jnp.float32),
            pltpu.VMEM_SHARED((HIST_ROWS, 16), jnp.float32),
        ],
    )
    return f(col32, ones16, zeros16)


def _spmm_body(C, xs_hbm, rowadj_hbm, colloc_hbm, zacc_hbm, out_hbm,
               radj_v, col_v, bufs, gsems, acc):
    cid = lax.axis_index("c")
    sid = lax.axis_index("s")
    pltpu.sync_copy(colloc_hbm.at[cid * 16 + sid], col_v)

    def gissue(jj, p):
        return pltpu.async_copy(xs_hbm.at[radj_v.at[jj]], bufs[p], gsems[p])

    for chunk in range(C):
        pltpu.sync_copy(zacc_hbm.at[pl.ds(sid * 328, 328)],
                        acc.at[pl.ds(sid * 328, 328)])
        pltpu.sync_copy(rowadj_hbm.at[chunk * 16 + sid], radj_v)
        plsc.subcore_barrier()
        gcps = [None] * NBA
        for j in range(4):
            gcps[j] = gissue(j, j % 4)
        for j in range(NBA):
            gcps[j].wait()
            pltpu.sync_copy(bufs[j % 4], acc.at[col_v.at[j]], add=True)
            if j + 4 < NBA:
                gcps[j + 4] = gissue(j + 4, j % 4)
        plsc.subcore_barrier()
        pltpu.sync_copy(
            acc.at[pl.ds(sid * 320, 320)],
            out_hbm.at[chunk].at[pl.ds(cid * HALF_N + sid * 320, 320)])
        plsc.subcore_barrier()


def _spmm(xs, rowadj, colloc, zacc, C):
    f = pl.kernel(
        functools.partial(_spmm_body, C),
        out_type=jax.ShapeDtypeStruct((C, 2 * HALF_N, 128), jnp.float32),
        mesh=_SC_MESH,
        scratch_types=[
            pltpu.VMEM((NBA, 128), jnp.int32),
            pltpu.VMEM((NBA, 128), jnp.int32),
            [pltpu.VMEM((128, 128), jnp.float32) for _ in range(4)],
            [pltpu.SemaphoreType.DMA for _ in range(4)],
            pltpu.VMEM_SHARED((ACC_ROWS, 128), jnp.float32),
        ],
    )
    return f(xs, rowadj, colloc, zacc)


def _decode_body(u_hbm, v_hbm, rp_hbm, cp_hbm, w2_hbm, b2_hbm, out_hbm,
                 rp_v, cp_v, ub0, ub1, vb0, vb1, w2_v, b2_v, out_v,
                 semu0, semu1, semv0, semv1):
    cid = lax.axis_index("c")
    sid = lax.axis_index("s")
    wid = cid * 16 + sid
    pltpu.sync_copy(rp_hbm.at[wid], rp_v)
    pltpu.sync_copy(cp_hbm.at[wid], cp_v)
    pltpu.sync_copy(w2_hbm, w2_v)
    pltpu.sync_copy(b2_hbm, b2_v)
    w2s = [w2_v[k] for k in range(HID_CH // 16)]
    it = _iota16()
    ubs = (ub0, ub1)
    vbs = (vb0, vb1)
    us = (semu0, semu1)
    vs = (semv0, semv1)

    def issue(jb, p):
        pltpu.async_copy(u_hbm.at[rp_v.at[jb]], ubs[p], us[p])
        pltpu.async_copy(v_hbm.at[cp_v.at[jb]], vbs[p], vs[p])

    def wait(jb, p):
        pltpu.make_async_copy(u_hbm.at[rp_v.at[jb]], ubs[p], us[p]).wait()
        pltpu.make_async_copy(v_hbm.at[cp_v.at[jb]], vbs[p], vs[p]).wait()

    def compute(jb, p):
        ubuf = ubs[p]
        vbuf = vbs[p]

        def edge(b, vec):
            acc = jnp.zeros((16,), jnp.float32)
            for k in range(HID_CH // 16):
                s = pl.ds(k * 16, 16)
                acc = acc + jnp.maximum(ubuf[b, s] + vbuf[b, s], 0.0) * w2s[k]
            for sh in (8, 4, 2, 1):
                acc = acc + _lane_shuffle(acc, it ^ sh)
            vec = jnp.where(it == (b & 15), acc, vec)

            @pl.when((b & 15) == 15)
            def _():
                out_v[pl.ds(jb * PB + (b // 16) * 16, 16)] = vec

            return vec

        lax.fori_loop(0, PB, edge, jnp.zeros((16,), jnp.float32))

    issue(0, 0)

    def pairstep(jp, _):
        j0 = jp * 2

        @pl.when(j0 + 1 < NPB)
        def _():
            issue(j0 + 1, 1)

        wait(j0, 0)
        compute(j0, 0)

        @pl.when(j0 + 2 < NPB)
        def _():
            issue(j0 + 2, 0)

        @pl.when(j0 + 1 < NPB)
        def _():
            wait(j0 + 1, 1)
            compute(j0 + 1, 1)

        return 0

    lax.fori_loop(0, (NPB + 1) // 2, pairstep, 0)

    b2s = b2_v[...]

    def sig(i, _):
        s = pl.ds(i * 16, 16)
        xx = out_v[s] + b2s
        out_v[s] = 1.0 / (1.0 + jnp.exp(-xx))
        return 0

    lax.fori_loop(0, PPW // 16, sig, 0)
    pltpu.sync_copy(out_v, out_hbm.at[pl.ds(wid * PPW, PPW)])


def _decode(u, v, rp_sc, cp_sc, w2, b2):
    f = pl.kernel(
        _decode_body,
        out_type=jax.ShapeDtypeStruct((32 * PPW,), jnp.float32),
        mesh=_SC_MESH,
        scratch_types=[
            pltpu.VMEM((NPB, PB), jnp.int32),
            pltpu.VMEM((NPB, PB), jnp.int32),
            pltpu.VMEM((PB, HID_CH), jnp.float32),
            pltpu.VMEM((PB, HID_CH), jnp.float32),
            pltpu.VMEM((PB, HID_CH), jnp.float32),
            pltpu.VMEM((PB, HID_CH), jnp.float32),
            pltpu.VMEM((HID_CH // 16, 16), jnp.float32),
            pltpu.VMEM((16,), jnp.float32),
            pltpu.VMEM((PPW,), jnp.float32),
            pltpu.SemaphoreType.DMA,
            pltpu.SemaphoreType.DMA,
            pltpu.SemaphoreType.DMA,
            pltpu.SemaphoreType.DMA,
        ],
    )
    return f(u, v, rp_sc, cp_sc, w2, b2)


_RB = 1000


def _mm1_body(x_ref, w_ref, h_ref, xs_ref, dis_ref):
    deg = h_ref[0, :, 0:1] + h_ref[1, :, 0:1]
    dis = jax.lax.rsqrt(deg)
    y = jnp.dot(x_ref[...], w_ref[...], preferred_element_type=jnp.float32)
    xs_ref[0] = y * dis
    dis_ref[...] = dis


def _mm1(x, W1, hist):
    nc = HID_CH // 128
    return pl.pallas_call(
        _mm1_body,
        grid=(N_NODES // _RB, nc),
        in_specs=[
            pl.BlockSpec((_RB, IN_CH), lambda i, c: (i, 0)),
            pl.BlockSpec((IN_CH, 128), lambda i, c: (0, c)),
            pl.BlockSpec((2, _RB, 16), lambda i, c: (0, i, 0)),
        ],
        out_specs=[
            pl.BlockSpec((1, _RB, 128), lambda i, c: (c, i, 0)),
            pl.BlockSpec((_RB, 1), lambda i, c: (i, 0)),
        ],
        out_shape=[
            jax.ShapeDtypeStruct((nc, N_NODES, 128), jnp.float32),
            jax.ShapeDtypeStruct((N_NODES, 1), jnp.float32),
        ],
    )(x, W1, hist)


def _mm2_body(acc_ref, w_ref, b_ref, dis_ref, xs_ref):
    dis = dis_ref[...]
    y = jnp.zeros((_RB, 128), jnp.float32)
    for k in range(HID_CH // 128):
        zk = jnp.maximum(acc_ref[k] * dis + b_ref[0, k * 128:(k + 1) * 128],
                         0.0)
        y = y + jnp.dot(zk, w_ref[k * 128:(k + 1) * 128, :],
                        preferred_element_type=jnp.float32)
    xs_ref[0] = y * dis


def _mm2(acc1, W2, b1, dis):
    nc = OUT_CH // 128
    return pl.pallas_call(
        _mm2_body,
        grid=(N_NODES // _RB, nc),
        in_specs=[
            pl.BlockSpec((HID_CH // 128, _RB, 128), lambda i, c: (0, i, 0)),
            pl.BlockSpec((HID_CH, 128), lambda i, c: (0, c)),
            pl.BlockSpec((1, HID_CH), lambda i, c: (0, 0)),
            pl.BlockSpec((_RB, 1), lambda i, c: (i, 0)),
        ],
        out_specs=pl.BlockSpec((1, _RB, 128), lambda i, c: (c, i, 0)),
        out_shape=jax.ShapeDtypeStruct((nc, N_NODES, 128), jnp.float32),
    )(acc1, W2, b1, dis)


def _mm3_body(acc_ref, wa_ref, wb_ref, b2_ref, blp_ref, dis_ref,
              u_ref, v_ref):
    dis = dis_ref[...]
    parts = [acc_ref[k] * dis + b2_ref[0, k * 128:(k + 1) * 128]
             for k in range(OUT_CH // 128)]
    z2 = jnp.concatenate(parts, axis=1)
    u_ref[...] = jnp.dot(z2, wa_ref[...],
                         preferred_element_type=jnp.float32) + blp_ref[...]
    v_ref[...] = jnp.dot(z2, wb_ref[...], preferred_element_type=jnp.float32)


def _mm3(acc2, W1a, W1b, b2, b1lp, dis):
    return pl.pallas_call(
        _mm3_body,
        grid=(N_NODES // _RB,),
        in_specs=[
            pl.BlockSpec((OUT_CH // 128, _RB, 128), lambda i: (0, i, 0)),
            pl.BlockSpec((OUT_CH, HID_CH), lambda i: (0, 0)),
            pl.BlockSpec((OUT_CH, HID_CH), lambda i: (0, 0)),
            pl.BlockSpec((1, OUT_CH), lambda i: (0, 0)),
            pl.BlockSpec((1, HID_CH), lambda i: (0, 0)),
            pl.BlockSpec((_RB, 1), lambda i: (i, 0)),
        ],
        out_specs=[
            pl.BlockSpec((_RB, HID_CH), lambda i: (i, 0)),
            pl.BlockSpec((_RB, HID_CH), lambda i: (i, 0)),
        ],
        out_shape=[
            jax.ShapeDtypeStruct((N_NODES, HID_CH), jnp.float32),
            jax.ShapeDtypeStruct((N_NODES, HID_CH), jnp.float32),
        ],
    )(acc2, W1a, W1b, b2, b1lp, dis)


def kernel(x, edge_index, pred_edge_index, conv1_W, conv1_b, conv2_W, conv2_b,
           lp_W1, lp_b1, lp_W2, lp_b2):
    loop = jnp.arange(N_NODES, dtype=jnp.int32)
    npad = E_TOT - E_REAL
    row = jnp.concatenate([edge_index[0].astype(jnp.int32), loop,
                           jnp.zeros((npad,), jnp.int32)])
    col = jnp.concatenate([edge_index[1].astype(jnp.int32), loop,
                           jnp.full((npad,), PAD_COL, jnp.int32)])
    col32 = col.reshape(32, E_TOT // 32 // 128, 128)
    nc1 = HID_CH // 128

    rowadj = (row.reshape(1, 16, NBA, 128)
              + (jnp.arange(nc1, dtype=jnp.int32) * N_NODES)[:, None, None,
                                                             None]
              ).reshape(nc1 * 16, NBA, 128)
    base = (jnp.arange(2, dtype=jnp.int32) * HALF_N)[:, None, None, None]
    colg = col.reshape(1, 16, NBA, 128)
    inhalf = (colg >= base) & (colg < base + HALF_N)
    colloc = jnp.where(inhalf, colg - base, TRASH_L).reshape(2 * 16, NBA, 128)
    ones16 = jnp.ones((128, 16), jnp.float32)
    zeros16 = jnp.zeros((HIST_ROWS, 16), jnp.float32)
    zacc = jnp.zeros((ACC_ROWS, 128), jnp.float32)

    hist = _deg(col32, ones16, zeros16)

    xs1, dis = _mm1(x, conv1_W, hist)
    acc1 = _spmm(xs1.reshape(-1, 128), rowadj, colloc, zacc, nc1)
    xs2 = _mm2(acc1, conv2_W, conv1_b.reshape(1, HID_CH), dis)
    acc2 = _spmm(xs2.reshape(-1, 128), rowadj[:(OUT_CH // 128) * 16], colloc,
                 zacc, OUT_CH // 128)
    u, v = _mm3(acc2, lp_W1[:OUT_CH], lp_W1[OUT_CH:],
                conv2_b.reshape(1, OUT_CH), lp_b1.reshape(1, HID_CH), dis)

    ppad = P_TOT - P_RAW
    rp = jnp.concatenate([pred_edge_index[0].astype(jnp.int32),
                          jnp.zeros((ppad,), jnp.int32)]).reshape(32, NPB, PB)
    cp = jnp.concatenate([pred_edge_index[1].astype(jnp.int32),
                          jnp.zeros((ppad,), jnp.int32)]).reshape(32, NPB, PB)
    w2 = lp_W2.reshape(HID_CH // 16, 16)
    b2 = jnp.broadcast_to(lp_b2, (16,))
    logits = _decode(u, v, rp, cp, w2, b2)
    return logits.reshape(P_TOT, 1)[:P_RAW]
